# baseline (speedup 1.0000x reference)
import functools

import jax
import jax.numpy as jnp
from jax import lax
from jax.experimental import pallas as pl
import jax.experimental.pallas.tpu as pltpu
from jax.experimental.pallas import tpu_sc as plsc

ROWS = 3
B, CP, HP, WP = 8, 8, 256, 256
H = HP + ROWS - 1
W = WP + ROWS - 1
C_CONST = 0.9
LORRIS = 0.25
HAMMER = 1.0
N = B * HP * WP
THRESH = C_CONST ** (1.0 / (H * W))

CHUNK = 32
NCHUNK = HP // CHUNK

B_TC = 6
B_SC = B - B_TC
NWORKER = 32
STRIP = HP * B_SC // NWORKER


def _tc_kernel(mu_ref, exp_ref, pr_ref, idx_ref, w1_ref, u0_ref, u1_ref,
               v_ref, b_ref, part_ref, acc_ref):
    bidx = pl.program_id(0)

    @pl.when(bidx == 0)
    def _init():
        for t in range(4):
            acc_ref[t] = 0.0

    cnt_acc = jnp.zeros((CHUNK, WP), dtype=jnp.float32)
    wid_acc = jnp.zeros((CHUNK, WP), dtype=jnp.float32)
    pen_acc = jnp.zeros((CHUNK, WP), dtype=jnp.float32)

    for chunk in range(NCHUNK):
        base = chunk * CHUNK
        rdc0 = (exp_ref[0, 0, base:base + CHUNK + 2, :]
                - mu_ref[0, 0, base:base + CHUNK + 2, :])
        rdc1 = (exp_ref[0, 1, base:base + CHUNK + 2, :]
                - mu_ref[0, 1, base:base + CHUNK + 2, :])
        t0 = [rdc0[:, j:j + WP] for j in range(ROWS)]
        t1 = [rdc1[:, j:j + WP] for j in range(ROWS)]
        idx = idx_ref[0, base:base + CHUNK, :]

        r = [jnp.full((CHUNK, WP), b_ref[0, c], dtype=jnp.float32)
             for c in range(4)]
        for ci in range(CP):
            p = pr_ref[0, ci, base:base + CHUNK, :]
            for c in range(4):
                r[c] += p * w1_ref[ci, c]

        piv0 = jnp.zeros((CHUNK, WP), dtype=jnp.float32)
        piv1 = jnp.zeros((CHUNK, WP), dtype=jnp.float32)
        for k in range(ROWS * ROWS):
            i, j = divmod(k, ROWS)
            s0 = t0[j][i:i + CHUNK, :]
            s1 = t1[j][i:i + CHUNK, :]
            m = idx == k
            piv0 = jnp.where(m, s0, piv0)
            piv1 = jnp.where(m, s1, piv1)
            for c in range(4):
                term = s0 * u0_ref[k, c] + s1 * u1_ref[k, c]
                r[c] += jnp.where(m, v_ref[k, c], term)

        r0, r1, r2, r3 = r
        full_in = (((piv0 - r0) >= 0.0) & ((piv1 - r1) >= 0.0)
                   & ((piv0 - r2) <= 0.0) & ((piv1 - r3) <= 0.0))
        cnt_acc += full_in.astype(jnp.float32)
        wid_acc += jnp.abs(r2 - r0) + jnp.abs(r3 - r1)
        over0 = jnp.maximum(piv0 - r2, 0.0)
        over1 = jnp.maximum(piv1 - r3, 0.0)
        under0 = jnp.maximum(r0 - piv0, 0.0)
        under1 = jnp.maximum(r1 - piv1, 0.0)
        pen_acc += (over0 * over0 + over1 * over1
                    + under0 * under0 + under1 * under1)

    acc_ref[0] += jnp.sum(cnt_acc)
    acc_ref[1] += jnp.sum(wid_acc)
    acc_ref[2] += jnp.sum(pen_acc)
    part_ref[0, 0] = acc_ref[0]
    part_ref[0, 1] = acc_ref[1]
    part_ref[0, 2] = acc_ref[2]
    part_ref[0, 3] = 0.0


def _sc_body(mu_hbm, exp_hbm, pr_hbm, idx_hbm, w_hbm, out_hbm,
             mu_v, exp_v, pr_v, idx_v, w_v, o_v):
    wid = lax.axis_index("c") * 16 + lax.axis_index("s")
    bb = wid // (NWORKER // B_SC)
    strip = wid % (NWORKER // B_SC)
    r0 = strip * STRIP

    for ch in range(2):
        pltpu.sync_copy(mu_hbm.at[bb * 2 + ch, pl.ds(r0, STRIP + 8), :],
                        mu_v.at[ch])
        pltpu.sync_copy(exp_hbm.at[bb * 2 + ch, pl.ds(r0, STRIP + 8), :],
                        exp_v.at[ch])
    for ci in range(CP):
        pltpu.sync_copy(pr_hbm.at[bb * CP + ci, pl.ds(r0, STRIP), :],
                        pr_v.at[ci])
    pltpu.sync_copy(idx_hbm.at[bb, pl.ds(r0, STRIP), :], idx_v)
    pltpu.sync_copy(w_hbm, w_v)

    wch = [w_v[pl.ds(16 * t, 16)] for t in range(9)]

    def wscal(i):
        return wch[i // 16][i % 16]

    u0s = [[wscal(4 * k + c) for c in range(4)] for k in range(9)]
    u1s = [[wscal(36 + 4 * k + c) for c in range(4)] for k in range(9)]
    vs = [[wscal(72 + 4 * k + c) for c in range(4)] for k in range(9)]
    w1s = [[wscal(108 + 4 * ci + c) for c in range(4)] for ci in range(CP)]
    bs = [wscal(140 + c) for c in range(4)]

    lane16 = lax.iota(jnp.int32, 16)

    def gat(ref, ch, row, x0):
        i0 = jnp.full((16,), ch, dtype=jnp.int32)
        i1 = jnp.full((16,), row, dtype=jnp.int32)
        return plsc.load_gather(ref, [i0, i1, x0 + lane16])

    def xblk_body(rl, xb, accs):
        cnt_a, wid_a, pen_a = accs
        xo = xb * 16
        idxv = idx_v[rl, pl.ds(xo, 16)]
        r = [jnp.full((16,), bs[c], dtype=jnp.float32) for c in range(4)]
        for ci in range(CP):
            p = pr_v[ci, rl, pl.ds(xo, 16)]
            for c in range(4):
                r[c] += p * w1s[ci][c]
        piv0 = jnp.zeros((16,), dtype=jnp.float32)
        piv1 = jnp.zeros((16,), dtype=jnp.float32)
        for k in range(ROWS * ROWS):
            i, j = divmod(k, ROWS)
            if j == 0:
                s0 = (exp_v[0, rl + i, pl.ds(xo, 16)]
                      - mu_v[0, rl + i, pl.ds(xo, 16)])
                s1 = (exp_v[1, rl + i, pl.ds(xo, 16)]
                      - mu_v[1, rl + i, pl.ds(xo, 16)])
            else:
                s0 = (gat(exp_v, 0, rl + i, xo + j)
                      - gat(mu_v, 0, rl + i, xo + j))
                s1 = (gat(exp_v, 1, rl + i, xo + j)
                      - gat(mu_v, 1, rl + i, xo + j))
            m = idxv == k
            piv0 = jnp.where(m, s0, piv0)
            piv1 = jnp.where(m, s1, piv1)
            for c in range(4):
                term = s0 * u0s[k][c] + s1 * u1s[k][c]
                r[c] += jnp.where(m, jnp.full((16,), vs[k][c],
                                              dtype=jnp.float32), term)
        r0_, r1_, r2_, r3_ = r
        full_in = (((piv0 - r0_) >= 0.0) & ((piv1 - r1_) >= 0.0)
                   & ((piv0 - r2_) <= 0.0) & ((piv1 - r3_) <= 0.0))
        cnt_a = cnt_a + full_in.astype(jnp.float32)
        wid_a = wid_a + jnp.abs(r2_ - r0_) + jnp.abs(r3_ - r1_)
        over0 = jnp.maximum(piv0 - r2_, 0.0)
        over1 = jnp.maximum(piv1 - r3_, 0.0)
        under0 = jnp.maximum(r0_ - piv0, 0.0)
        under1 = jnp.maximum(r1_ - piv1, 0.0)
        pen_a = pen_a + (over0 * over0 + over1 * over1
                         + under0 * under0 + under1 * under1)
        return cnt_a, wid_a, pen_a

    def row_body(rl, accs):
        return lax.fori_loop(
            0, WP // 16, lambda xb, a: xblk_body(rl, xb, a), accs)

    zero = jnp.zeros((16,), dtype=jnp.float32)
    cnt_a, wid_a, pen_a = lax.fori_loop(0, STRIP, row_body,
                                        (zero, zero, zero))
    lane = lax.iota(jnp.int32, 16)
    out_vec = jnp.where(lane == 0, jnp.sum(cnt_a),
                        jnp.where(lane == 1, jnp.sum(wid_a),
                                  jnp.where(lane == 2, jnp.sum(pen_a), 0.0)))
    o_v[0, :] = out_vec.astype(jnp.float32)
    zv = jnp.zeros((16,), dtype=jnp.float32)
    for t in range(1, 8):
        o_v[t, :] = zv
    pltpu.sync_copy(o_v, out_hbm.at[pl.ds(wid * 8, 8), :])


@functools.partial(jax.jit)
def _run(actual_mu, actual_pruning, expected, W1, W2, b, idx3):
    W2r = W2.reshape(ROWS * ROWS, 4, 4)
    U0 = W2r[:, 0, :] + W2r[:, 2, :]
    U1 = W2r[:, 1, :] + W2r[:, 3, :]
    V = W2r[:, 0, :] + W2r[:, 1, :] - W2r[:, 2, :] - W2r[:, 3, :]
    b2 = b.reshape(1, 4)

    smem = pl.BlockSpec(memory_space=pltpu.SMEM)
    tc_part = pl.pallas_call(
        _tc_kernel,
        grid=(B_TC,),
        in_specs=[
            pl.BlockSpec((1, 2, H, W), lambda i: (i, 0, 0, 0)),
            pl.BlockSpec((1, 2, H, W), lambda i: (i, 0, 0, 0)),
            pl.BlockSpec((1, CP, HP, WP), lambda i: (i, 0, 0, 0)),
            pl.BlockSpec((1, HP, WP), lambda i: (i, 0, 0)),
            smem, smem, smem, smem, smem,
        ],
        out_specs=pl.BlockSpec(memory_space=pltpu.SMEM),
        out_shape=jax.ShapeDtypeStruct((1, 4), jnp.float32),
        scratch_shapes=[pltpu.SMEM((4,), jnp.float32)],
        compiler_params=pltpu.CompilerParams(
            dimension_semantics=("arbitrary",)),
    )(actual_mu, expected, actual_pruning, idx3, W1, U0, U1, V, b2)

    mu_sc = jnp.pad(actual_mu[B_TC:].reshape(B_SC * 2, H, W),
                    ((0, 0), (0, 6), (0, 0)))
    exp_sc = jnp.pad(expected[B_TC:].reshape(B_SC * 2, H, W),
                     ((0, 0), (0, 6), (0, 0)))
    pr_sc = actual_pruning[B_TC:].reshape(B_SC * CP, HP, WP)
    idx_sc = idx3[B_TC:]
    wflat = jnp.concatenate([U0.ravel(), U1.ravel(), V.ravel(),
                             W1.ravel(), b]).astype(jnp.float32)

    mesh = plsc.VectorSubcoreMesh(core_axis_name="c", subcore_axis_name="s")
    sc_part = pl.kernel(
        _sc_body,
        mesh=mesh,
        out_type=jax.ShapeDtypeStruct((NWORKER * 8, 16), jnp.float32),
        scratch_types=[
            pltpu.VMEM((2, STRIP + 8, W), jnp.float32),
            pltpu.VMEM((2, STRIP + 8, W), jnp.float32),
            pltpu.VMEM((CP, STRIP, WP), jnp.float32),
            pltpu.VMEM((STRIP, WP), jnp.int32),
            pltpu.VMEM((144,), jnp.float32),
            pltpu.VMEM((8, 16), jnp.float32),
        ],
        compiler_params=pltpu.CompilerParams(use_tc_tiling_on_sc=False,
                                             needs_layout_passes=False),
    )(mu_sc, exp_sc, pr_sc, idx_sc, wflat)

    cnt = tc_part[0, 0] + jnp.sum(sc_part[:, 0])
    wid = tc_part[0, 1] + jnp.sum(sc_part[:, 1])
    pen = tc_part[0, 2] + jnp.sum(sc_part[:, 2])
    p_in = cnt * (1.0 / N)
    penalty = pen * (HAMMER / (2.0 * N))
    return LORRIS * wid + jnp.where(p_in < THRESH, penalty, 0.0)


def kernel(actual_mu, actual_pruning, expected, W1, W2, b, index_choice):
    idx3 = index_choice.reshape(B, HP, WP)
    return _run(actual_mu, actual_pruning, expected, W1, W2, b, idx3)


# speedup vs baseline: 2.0316x; 2.0316x over previous
import functools

import jax
import jax.numpy as jnp
from jax.experimental import pallas as pl
import jax.experimental.pallas.tpu as pltpu

ROWS = 3
B, CP, HP, WP = 8, 8, 256, 256
H = HP + ROWS - 1
W = WP + ROWS - 1
C_CONST = 0.9
LORRIS = 0.25
HAMMER = 1.0
N = B * HP * WP
THRESH = C_CONST ** (1.0 / (H * W))

CHUNK = 32
NCHUNK = HP // CHUNK


def _loss_kernel(mu_ref, exp_ref, pr_ref, idx_ref, w1_ref, u0_ref, u1_ref,
                 v_ref, b_ref, loss_ref, acc_ref):
    bidx = pl.program_id(0)

    @pl.when(bidx == 0)
    def _init():
        for t in range(4):
            acc_ref[t] = 0.0

    cnt_acc = jnp.zeros((CHUNK, WP), dtype=jnp.float32)
    wid_acc = jnp.zeros((CHUNK, WP), dtype=jnp.float32)
    pen_acc = jnp.zeros((CHUNK, WP), dtype=jnp.float32)

    for chunk in range(NCHUNK):
        base = chunk * CHUNK
        rdc0 = (exp_ref[0, 0, base:base + CHUNK + 2, :]
                - mu_ref[0, 0, base:base + CHUNK + 2, :])
        rdc1 = (exp_ref[0, 1, base:base + CHUNK + 2, :]
                - mu_ref[0, 1, base:base + CHUNK + 2, :])
        t0 = [rdc0[:, j:j + WP] for j in range(ROWS)]
        t1 = [rdc1[:, j:j + WP] for j in range(ROWS)]
        idx = idx_ref[0, base:base + CHUNK, :]

        r = [jnp.full((CHUNK, WP), b_ref[0, c], dtype=jnp.float32)
             for c in range(4)]
        for ci in range(CP):
            p = pr_ref[0, ci, base:base + CHUNK, :]
            for c in range(4):
                r[c] += p * w1_ref[ci, c]

        piv0 = jnp.zeros((CHUNK, WP), dtype=jnp.float32)
        piv1 = jnp.zeros((CHUNK, WP), dtype=jnp.float32)
        for k in range(ROWS * ROWS):
            i, j = divmod(k, ROWS)
            s0 = t0[j][i:i + CHUNK, :]
            s1 = t1[j][i:i + CHUNK, :]
            m = idx == k
            piv0 = jnp.where(m, s0, piv0)
            piv1 = jnp.where(m, s1, piv1)
            for c in range(4):
                term = s0 * u0_ref[k, c] + s1 * u1_ref[k, c]
                r[c] += jnp.where(m, v_ref[k, c], term)

        r0, r1, r2, r3 = r
        full_in = (((piv0 - r0) >= 0.0) & ((piv1 - r1) >= 0.0)
                   & ((piv0 - r2) <= 0.0) & ((piv1 - r3) <= 0.0))
        cnt_acc += full_in.astype(jnp.float32)
        wid_acc += jnp.abs(r2 - r0) + jnp.abs(r3 - r1)
        over0 = jnp.maximum(piv0 - r2, 0.0)
        over1 = jnp.maximum(piv1 - r3, 0.0)
        under0 = jnp.maximum(r0 - piv0, 0.0)
        under1 = jnp.maximum(r1 - piv1, 0.0)
        pen_acc += (over0 * over0 + over1 * over1
                    + under0 * under0 + under1 * under1)

    acc_ref[0] += jnp.sum(cnt_acc)
    acc_ref[1] += jnp.sum(wid_acc)
    acc_ref[2] += jnp.sum(pen_acc)

    @pl.when(bidx == B - 1)
    def _fini():
        p_in = acc_ref[0] * (1.0 / N)
        penalty = acc_ref[2] * (HAMMER / (2.0 * N))
        loss = LORRIS * acc_ref[1] + jnp.where(p_in < THRESH, penalty, 0.0)
        loss_ref[0, 0] = loss


@functools.partial(jax.jit)
def _run(actual_mu, actual_pruning, expected, W1, W2, b, idx3):
    W2r = W2.reshape(ROWS * ROWS, 4, 4)
    U0 = W2r[:, 0, :] + W2r[:, 2, :]
    U1 = W2r[:, 1, :] + W2r[:, 3, :]
    V = W2r[:, 0, :] + W2r[:, 1, :] - W2r[:, 2, :] - W2r[:, 3, :]
    b2 = b.reshape(1, 4)

    smem = pl.BlockSpec(memory_space=pltpu.SMEM)
    out = pl.pallas_call(
        _loss_kernel,
        grid=(B,),
        in_specs=[
            pl.BlockSpec((1, 2, H, W), lambda i: (i, 0, 0, 0)),
            pl.BlockSpec((1, 2, H, W), lambda i: (i, 0, 0, 0)),
            pl.BlockSpec((1, CP, HP, WP), lambda i: (i, 0, 0, 0)),
            pl.BlockSpec((1, HP, WP), lambda i: (i, 0, 0)),
            smem, smem, smem, smem, smem,
        ],
        out_specs=pl.BlockSpec(memory_space=pltpu.SMEM),
        out_shape=jax.ShapeDtypeStruct((1, 1), jnp.float32),
        scratch_shapes=[pltpu.SMEM((4,), jnp.float32)],
        compiler_params=pltpu.CompilerParams(
            dimension_semantics=("arbitrary",)),
    )(actual_mu, expected, actual_pruning, idx3, W1, U0, U1, V, b2)
    return out.reshape(())


def kernel(actual_mu, actual_pruning, expected, W1, W2, b, index_choice):
    idx3 = index_choice.reshape(B, HP, WP)
    return _run(actual_mu, actual_pruning, expected, W1, W2, b, idx3)
